# baseline (speedup 1.0000x reference)
import functools

import jax
import jax.numpy as jnp
from jax import lax
from jax.experimental import pallas as pl
from jax.experimental.pallas import tpu as pltpu
from jax.experimental.pallas import tpu_sc as plsc

NC = 2
NS = 16
NW = NC * NS
B = 16384
D = 128
BPW = B // NW

_mesh = plsc.VectorSubcoreMesh(core_axis_name="c", subcore_axis_name="s")


@functools.partial(
    pl.kernel,
    out_type=jax.ShapeDtypeStruct((B, D), jnp.float32),
    mesh=_mesh,
    scratch_types=[
        pltpu.VMEM((BPW,), jnp.int32),
        pltpu.VMEM((BPW, D), jnp.float32),
    ]
    + [pltpu.SemaphoreType.DMA] * 6,
)
def _gather_kernel(ids_hbm, table_hbm, out_hbm, idx_v, rows_v, *sems):
    isems, gsem, wsem = sems[:4], sems[4], sems[5]
    wid = lax.axis_index("s") * NC + lax.axis_index("c")
    nchunk = 4
    ck = BPW // nchunk
    icopies = [
        pltpu.async_copy(
            ids_hbm.at[pl.ds(wid * BPW + j * ck, ck)],
            idx_v.at[pl.ds(j * ck, ck)],
            isems[j],
        )
        for j in range(nchunk)
    ]
    gathers = []
    for j in range(nchunk):
        icopies[j].wait()
        gathers.append(
            pltpu.async_copy(
                table_hbm.at[idx_v.at[pl.ds(j * ck, ck)]],
                rows_v.at[pl.ds(j * ck, ck)],
                gsem,
            )
        )
    for j in range(nchunk):
        gathers[j].wait()
    pltpu.async_copy(rows_v, out_hbm.at[pl.ds(wid * BPW, BPW)], wsem).wait()


def kernel(node_ids, loc0, loc1, node_embeds):
    return _gather_kernel(node_ids.astype(jnp.int32), node_embeds)


# speedup vs baseline: 1.0085x; 1.0085x over previous
import functools

import jax
import jax.numpy as jnp
from jax import lax
from jax.experimental import pallas as pl
from jax.experimental.pallas import tpu as pltpu
from jax.experimental.pallas import tpu_sc as plsc

NC = 2
NS = 16
NW = NC * NS
B = 16384
D = 128
BPW = B // NW

_mesh = plsc.VectorSubcoreMesh(core_axis_name="c", subcore_axis_name="s")


@functools.partial(
    pl.kernel,
    out_type=jax.ShapeDtypeStruct((B, D), jnp.float32),
    mesh=_mesh,
    scratch_types=[
        pltpu.VMEM((BPW,), jnp.int32),
        pltpu.VMEM((BPW, D), jnp.float32),
        pltpu.SemaphoreType.DMA,
    ],
)
def _gather_kernel(ids_hbm, table_hbm, out_hbm, idx_v, rows_v, sem):
    wid = lax.axis_index("s") * NC + lax.axis_index("c")
    pltpu.sync_copy(ids_hbm.at[pl.ds(wid * BPW, BPW)], idx_v)
    pltpu.async_copy(table_hbm.at[idx_v], rows_v, sem).wait()
    pltpu.sync_copy(rows_v, out_hbm.at[pl.ds(wid * BPW, BPW)])


def kernel(node_ids, loc0, loc1, node_embeds):
    return _gather_kernel(node_ids.astype(jnp.int32), node_embeds)
